# Initial kernel scaffold; baseline (speedup 1.0000x reference)
import functools

import jax
import jax.numpy as jnp
from jax import lax
from jax.experimental import pallas as pl
from jax.experimental.pallas import tpu as pltpu
from jax.experimental.pallas import tpu_sc as plsc

N = 10000
E = 320000
NET = 4
D_IN = 128
H = 64
HH = H // 2

LANES = 128
EROWS = 2528
EPAD = EROWS * LANES
R1 = EROWS // 16
R2 = EROWS // 32
NPAD = 10048
RPT = NPAD // 16

_MESH = plsc.VectorSubcoreMesh(core_axis_name="c", subcore_axis_name="s")


def _build_deg_kernel():
    @functools.partial(
        pl.kernel,
        out_type=[jax.ShapeDtypeStruct((NPAD, 1), jnp.float32)] * 8,
        mesh=_MESH,
        scratch_types=[
            pltpu.VMEM((R2, LANES), jnp.int32),
            pltpu.VMEM((LANES, 1), jnp.float32),
            pltpu.VMEM((RPT, 1), jnp.float32),
        ] + [pltpu.VMEM_SHARED((NPAD, 1), jnp.float32)] * NET,
    )
    def deg_kernel(*refs):
        dsts = refs[0:4]
        ones_h, zeros_h = refs[4], refs[5]
        outs = refs[6:14]
        idx_v, ones_v, zb = refs[14:17]
        accs = refs[17:21]
        cid = lax.axis_index("c")
        sid = lax.axis_index("s")
        wid = cid * 16 + sid
        pltpu.sync_copy(ones_h, ones_v)
        pltpu.sync_copy(zeros_h, zb)
        for ni in range(NET):
            pltpu.sync_copy(zb, accs[ni].at[pl.ds(sid * RPT, RPT)])
        plsc.subcore_barrier()
        for ni in range(NET):
            pltpu.sync_copy(dsts[ni].at[pl.ds(wid * R2, R2)], idx_v)
            acc = accs[ni]

            def chunk(j, carry):
                pltpu.sync_copy(ones_v, acc.at[idx_v.at[j]], add=True)
                return carry

            lax.fori_loop(0, R2, chunk, 0)
        plsc.subcore_barrier()
        for c in range(2):
            @pl.when(cid == c)
            def _():
                for ni in range(NET):
                    sl = pl.ds(sid * RPT, RPT)
                    pltpu.sync_copy(accs[ni].at[sl], outs[c * 4 + ni].at[sl])

    return deg_kernel


def _build_conv1_kernel():
    @functools.partial(
        pl.kernel,
        out_type=[jax.ShapeDtypeStruct((NPAD, HH), jnp.float32)] * 8,
        mesh=_MESH,
        scratch_types=[
            pltpu.VMEM((R1, LANES), jnp.int32),
            pltpu.VMEM((R1, LANES), jnp.int32),
            pltpu.VMEM((LANES, HH), jnp.float32),
            pltpu.VMEM((RPT, HH), jnp.float32),
        ] + [pltpu.VMEM_SHARED((NPAD, HH), jnp.float32)] * NET,
    )
    def conv1_kernel(*refs):
        srcs = refs[0:4]
        dsts = refs[4:8]
        ztabs = refs[8:16]
        zeros_h = refs[16]
        outs = refs[17:25]
        src_v, dst_v, rows, zb = refs[25:29]
        accs = refs[29:33]
        cid = lax.axis_index("c")
        sid = lax.axis_index("s")
        pltpu.sync_copy(zeros_h, zb)
        for ni in range(NET):
            pltpu.sync_copy(zb, accs[ni].at[pl.ds(sid * RPT, RPT)])
        plsc.subcore_barrier()
        for c in range(2):
            @pl.when(cid == c)
            def _():
                for ni in range(NET):
                    pltpu.sync_copy(srcs[ni].at[pl.ds(sid * R1, R1)], src_v)
                    pltpu.sync_copy(dsts[ni].at[pl.ds(sid * R1, R1)], dst_v)
                    ztab = ztabs[c * 4 + ni]
                    acc = accs[ni]

                    def chunk(j, carry):
                        pltpu.sync_copy(ztab.at[src_v.at[j]], rows)
                        pltpu.sync_copy(rows, acc.at[dst_v.at[j]], add=True)
                        return carry

                    lax.fori_loop(0, R1, chunk, 0)
        plsc.subcore_barrier()
        for c in range(2):
            @pl.when(cid == c)
            def _():
                for ni in range(NET):
                    sl = pl.ds(sid * RPT, RPT)
                    pltpu.sync_copy(accs[ni].at[sl], outs[c * 4 + ni].at[sl])

    return conv1_kernel


def _build_conv2_kernel():
    @functools.partial(
        pl.kernel,
        out_type=[jax.ShapeDtypeStruct((NPAD, 2), jnp.float32)] * 8,
        mesh=_MESH,
        scratch_types=[
            pltpu.VMEM((R2, LANES), jnp.int32),
            pltpu.VMEM((R2, LANES), jnp.int32),
            pltpu.VMEM((LANES, 2), jnp.float32),
            pltpu.VMEM((RPT, 2), jnp.float32),
        ] + [pltpu.VMEM_SHARED((NPAD, 2), jnp.float32)] * NET,
    )
    def conv2_kernel(*refs):
        srcs = refs[0:4]
        dsts = refs[4:8]
        ztabs = refs[8:12]
        zeros_h = refs[12]
        outs = refs[13:21]
        src_v, dst_v, rows, zb = refs[21:25]
        accs = refs[25:29]
        cid = lax.axis_index("c")
        sid = lax.axis_index("s")
        wid = cid * 16 + sid
        pltpu.sync_copy(zeros_h, zb)
        for ni in range(NET):
            pltpu.sync_copy(zb, accs[ni].at[pl.ds(sid * RPT, RPT)])
        plsc.subcore_barrier()
        for ni in range(NET):
            pltpu.sync_copy(srcs[ni].at[pl.ds(wid * R2, R2)], src_v)
            pltpu.sync_copy(dsts[ni].at[pl.ds(wid * R2, R2)], dst_v)
            ztab = ztabs[ni]
            acc = accs[ni]

            def chunk(j, carry):
                pltpu.sync_copy(ztab.at[src_v.at[j]], rows)
                pltpu.sync_copy(rows, acc.at[dst_v.at[j]], add=True)
                return carry

            lax.fori_loop(0, R2, chunk, 0)
        plsc.subcore_barrier()
        for c in range(2):
            @pl.when(cid == c)
            def _():
                for ni in range(NET):
                    sl = pl.ds(sid * RPT, RPT)
                    pltpu.sync_copy(accs[ni].at[sl], outs[c * 4 + ni].at[sl])

    return conv2_kernel


_DEG_KERNEL = _build_deg_kernel()
_CONV1_KERNEL = _build_conv1_kernel()
_CONV2_KERNEL = _build_conv2_kernel()


def _tc1(flatten, W1):
    def body(f_ref, w_ref, o_ref):
        o_ref[0] = jnp.dot(f_ref[...], w_ref[0], preferred_element_type=jnp.float32)

    return pl.pallas_call(
        body,
        grid=(NET,),
        in_specs=[pl.BlockSpec((N, D_IN), lambda i: (0, 0)),
                  pl.BlockSpec((1, D_IN, H), lambda i: (i, 0, 0))],
        out_specs=pl.BlockSpec((1, N, H), lambda i: (i, 0, 0)),
        out_shape=jax.ShapeDtypeStruct((NET, N, H), jnp.float32),
    )(flatten, W1)


def _tc2(degA, degB, x1):
    def body(dA, dB, x_ref, z_ref, di_ref):
        deg = dA[0] + dB[0] + 2.0
        di = lax.rsqrt(deg)
        z_ref[0, 0] = x_ref[0] * di[:, None]
        di_ref[0] = di

    return pl.pallas_call(
        body,
        grid=(NET, 2),
        in_specs=[pl.BlockSpec((1, N), lambda i, c: (i, 0)),
                  pl.BlockSpec((1, N), lambda i, c: (i, 0)),
                  pl.BlockSpec((1, N, HH), lambda i, c: (i, 0, c))],
        out_specs=[pl.BlockSpec((1, 1, N, HH), lambda i, c: (c, i, 0, 0)),
                   pl.BlockSpec((1, N), lambda i, c: (i, 0))],
        out_shape=[jax.ShapeDtypeStruct((2, NET, N, HH), jnp.float32),
                   jax.ShapeDtypeStruct((NET, N), jnp.float32)],
    )(degA, degB, x1)


def _tc3(accA, accB, x1, dinv, b1, gamma, beta, W2):
    def body(aA, aB, x1_ref, di_ref, b1_ref, g_ref, be_ref, w2_ref, y2_ref, z2_ref):
        di = di_ref[0]
        agg = jnp.concatenate([aA[0], aB[0]], axis=1)
        x = agg * di[:, None] + x1_ref[0] * (2.0 * di * di)[:, None] + b1_ref[0][None, :]
        x = jnp.maximum(x, 0.0)
        mean = jnp.mean(x, axis=0, keepdims=True)
        var = jnp.mean((x - mean) ** 2, axis=0, keepdims=True)
        x = (x - mean) * lax.rsqrt(var + 1e-5) * g_ref[0][None, :] + be_ref[0][None, :]
        y2 = jnp.dot(x, w2_ref[0], preferred_element_type=jnp.float32)
        y2_ref[0] = y2
        z2_ref[0] = y2 * di[:, None]

    return pl.pallas_call(
        body,
        grid=(NET,),
        in_specs=[pl.BlockSpec((1, N, HH), lambda i: (i, 0, 0)),
                  pl.BlockSpec((1, N, HH), lambda i: (i, 0, 0)),
                  pl.BlockSpec((1, N, H), lambda i: (i, 0, 0)),
                  pl.BlockSpec((1, N), lambda i: (i, 0)),
                  pl.BlockSpec((1, H), lambda i: (i, 0)),
                  pl.BlockSpec((1, H), lambda i: (i, 0)),
                  pl.BlockSpec((1, H), lambda i: (i, 0)),
                  pl.BlockSpec((1, H, 2), lambda i: (i, 0, 0))],
        out_specs=[pl.BlockSpec((1, N, 2), lambda i: (i, 0, 0)),
                   pl.BlockSpec((1, N, 2), lambda i: (i, 0, 0))],
        out_shape=[jax.ShapeDtypeStruct((NET, N, 2), jnp.float32)] * 2,
    )(accA, accB, x1, dinv, b1, gamma, beta, W2)


def _tc4(a2A, a2B, y2, dinv, b2, moe, gW, gb2):
    def body(aA, aB, y2_ref, di_ref, b2_ref, moe_ref, gW_ref, gb_ref, outl, outp):
        logits = jnp.dot(moe_ref[...], gW_ref[...],
                         preferred_element_type=jnp.float32) + gb_ref[0][None, :]
        m = jnp.max(logits, axis=1, keepdims=True)
        eg = jnp.exp(logits - m)
        g = eg / jnp.sum(eg, axis=1, keepdims=True)
        accl = jnp.zeros((N, 2), jnp.float32)
        accp = jnp.zeros((N, 2), jnp.float32)
        for i in range(NET):
            di = di_ref[i]
            o = ((aA[i] + aB[i]) * di[:, None]
                 + y2_ref[i] * (2.0 * di * di)[:, None] + b2_ref[i][None, :])
            mm = jnp.max(o, axis=1, keepdims=True)
            lse = mm + jnp.log(jnp.sum(jnp.exp(o - mm), axis=1, keepdims=True))
            lp = o - lse
            pp = jnp.exp(lp)
            w = g[:, i][:, None]
            accl = accl + w * lp
            accp = accp + w * pp
        outl[...] = accl
        outp[...] = accp

    return pl.pallas_call(
        body,
        out_shape=[jax.ShapeDtypeStruct((N, 2), jnp.float32)] * 2,
    )(a2A, a2B, y2, dinv, b2, moe, gW, gb2)


def kernel(features, moe_features, networks, flatten, W1, b1, gamma, beta,
           W2, b2, gW, gb):
    nets = networks.astype(jnp.int32)
    src = jnp.concatenate(
        [nets[:, 0, :], jnp.zeros((NET, EPAD - E), jnp.int32)], axis=1
    ).reshape(NET, EROWS, LANES)
    dst = jnp.concatenate(
        [nets[:, 1, :], jnp.full((NET, EPAD - E), N, jnp.int32)], axis=1
    ).reshape(NET, EROWS, LANES)
    src_l = [src[i] for i in range(NET)]
    dst_l = [dst[i] for i in range(NET)]
    ones128 = jnp.ones((LANES, 1), jnp.float32)
    zeros1 = jnp.zeros((RPT, 1), jnp.float32)
    zeros2 = jnp.zeros((RPT, 2), jnp.float32)
    zerosH = jnp.zeros((RPT, HH), jnp.float32)

    deg_outs = _DEG_KERNEL(*dst_l, ones128, zeros1)
    degA = jnp.stack([deg_outs[i][:N, 0] for i in range(NET)])
    degB = jnp.stack([deg_outs[4 + i][:N, 0] for i in range(NET)])

    x1 = _tc1(flatten, W1)
    z1, dinv = _tc2(degA, degB, x1)

    z1_l = [z1[c, i] for c in range(2) for i in range(NET)]
    a1 = _CONV1_KERNEL(*src_l, *dst_l, *z1_l, zerosH)
    accA = jnp.stack([a1[i][:N] for i in range(NET)])
    accB = jnp.stack([a1[4 + i][:N] for i in range(NET)])

    y2, z2 = _tc3(accA, accB, x1, dinv, b1, gamma, beta, W2)
    z2_l = [z2[i] for i in range(NET)]
    a2 = _CONV2_KERNEL(*src_l, *dst_l, *z2_l, zeros2)
    a2A = jnp.stack([a2[i][:N] for i in range(NET)])
    a2B = jnp.stack([a2[4 + i][:N] for i in range(NET)])

    res = _tc4(a2A, a2B, y2, dinv, b2, moe_features, gW,
               jnp.reshape(gb, (1, NET)))
    return (res[0], res[1])


# speedup vs baseline: 14.2578x; 14.2578x over previous
import functools

import jax
import jax.numpy as jnp
from jax import lax
from jax.experimental import pallas as pl
from jax.experimental.pallas import tpu as pltpu
from jax.experimental.pallas import tpu_sc as plsc

N = 10000
E = 320000
NET = 4
D_IN = 128
H = 64
HH = H // 2

LANES = 128
EROWS = 2560
EPAD = EROWS * LANES
R1 = EROWS // 16
R2 = EROWS // 32
NPAD = 10112
RPT = NPAD // 16

def _mesh():
    return plsc.VectorSubcoreMesh(core_axis_name="c", subcore_axis_name="s")


@functools.cache
def _build_deg_kernel():
    @functools.partial(
        pl.kernel,
        out_type=[jax.ShapeDtypeStruct((NPAD, HH), jnp.float32)] * 2,
        mesh=_mesh(),
        compiler_params=pltpu.CompilerParams(use_tc_tiling_on_sc=False),
        scratch_types=[
            pltpu.VMEM((R2, LANES), jnp.int32),
            pltpu.VMEM((LANES, HH), jnp.float32),
            pltpu.VMEM((RPT, HH), jnp.float32),
            pltpu.VMEM_SHARED((NPAD, HH), jnp.float32),
        ],
    )
    def deg_kernel(*refs):
        dsts = refs[0:4]
        onesoh = refs[4:8]
        zeros_h = refs[8]
        outs = refs[9:11]
        idx_v, ones_v, zb, acc = refs[11:15]
        cid = lax.axis_index("c")
        sid = lax.axis_index("s")
        wid = cid * 16 + sid
        pltpu.sync_copy(zeros_h, zb)
        pltpu.sync_copy(zb, acc.at[pl.ds(sid * RPT, RPT)])
        plsc.subcore_barrier()
        for ni in range(NET):
            pltpu.sync_copy(onesoh[ni], ones_v)
            pltpu.sync_copy(dsts[ni].at[pl.ds(wid * R2, R2)], idx_v)

            def chunk(j, carry):
                pltpu.sync_copy(ones_v, acc.at[idx_v.at[j]], add=True)
                return carry

            lax.fori_loop(0, R2, chunk, 0)
        plsc.subcore_barrier()
        for c in range(2):
            @pl.when(cid == c)
            def _():
                sl = pl.ds(sid * RPT, RPT)
                pltpu.sync_copy(acc.at[sl], outs[c].at[sl])

    return deg_kernel


@functools.cache
def _build_conv1_kernel():
    @functools.partial(
        pl.kernel,
        out_type=[jax.ShapeDtypeStruct((NPAD, HH), jnp.float32)] * 8,
        mesh=_mesh(),
        compiler_params=pltpu.CompilerParams(use_tc_tiling_on_sc=False),
        scratch_types=[
            pltpu.VMEM((R1, LANES), jnp.int32),
            pltpu.VMEM((R1, LANES), jnp.int32),
            pltpu.VMEM((LANES, HH), jnp.float32),
            pltpu.VMEM((RPT, HH), jnp.float32),
        ] + [pltpu.VMEM_SHARED((NPAD, HH), jnp.float32)] * 2,
    )
    def conv1_kernel(*refs):
        srcs = refs[0:4]
        dsts = refs[4:8]
        ztabs = refs[8:16]
        zeros_h = refs[16]
        outs = refs[17:25]
        src_v, dst_v, rows, zb = refs[25:29]
        accs = refs[29:31]
        cid = lax.axis_index("c")
        sid = lax.axis_index("s")
        pltpu.sync_copy(zeros_h, zb)
        for g in range(2):
            for k in range(2):
                pltpu.sync_copy(zb, accs[k].at[pl.ds(sid * RPT, RPT)])
            plsc.subcore_barrier()
            for c in range(2):
                @pl.when(cid == c)
                def _():
                    for k in range(2):
                        ni = 2 * g + k
                        pltpu.sync_copy(srcs[ni].at[pl.ds(sid * R1, R1)], src_v)
                        pltpu.sync_copy(dsts[ni].at[pl.ds(sid * R1, R1)], dst_v)
                        ztab = ztabs[c * 4 + ni]
                        acc = accs[k]

                        def chunk(j, carry):
                            pltpu.sync_copy(ztab.at[src_v.at[j]], rows)
                            pltpu.sync_copy(rows, acc.at[dst_v.at[j]], add=True)
                            return carry

                        lax.fori_loop(0, R1, chunk, 0)
            plsc.subcore_barrier()
            for c in range(2):
                @pl.when(cid == c)
                def _():
                    for k in range(2):
                        ni = 2 * g + k
                        sl = pl.ds(sid * RPT, RPT)
                        pltpu.sync_copy(accs[k].at[sl], outs[c * 4 + ni].at[sl])

    return conv1_kernel


@functools.cache
def _build_conv2_kernel():
    @functools.partial(
        pl.kernel,
        out_type=[jax.ShapeDtypeStruct((NPAD, HH), jnp.float32)] * 2,
        mesh=_mesh(),
        compiler_params=pltpu.CompilerParams(use_tc_tiling_on_sc=False),
        scratch_types=[
            pltpu.VMEM((R2, LANES), jnp.int32),
            pltpu.VMEM((R2, LANES), jnp.int32),
            pltpu.VMEM((LANES, HH), jnp.float32),
            pltpu.VMEM((RPT, HH), jnp.float32),
            pltpu.VMEM_SHARED((NPAD, HH), jnp.float32),
        ],
    )
    def conv2_kernel(*refs):
        srcs = refs[0:4]
        dsts = refs[4:8]
        ztabs = refs[8:12]
        zeros_h = refs[12]
        outs = refs[13:15]
        src_v, dst_v, rows, zb, acc = refs[15:20]
        cid = lax.axis_index("c")
        sid = lax.axis_index("s")
        wid = cid * 16 + sid
        pltpu.sync_copy(zeros_h, zb)
        pltpu.sync_copy(zb, acc.at[pl.ds(sid * RPT, RPT)])
        plsc.subcore_barrier()
        for ni in range(NET):
            pltpu.sync_copy(srcs[ni].at[pl.ds(wid * R2, R2)], src_v)
            pltpu.sync_copy(dsts[ni].at[pl.ds(wid * R2, R2)], dst_v)
            ztab = ztabs[ni]

            def chunk(j, carry):
                pltpu.sync_copy(ztab.at[src_v.at[j]], rows)
                pltpu.sync_copy(rows, acc.at[dst_v.at[j]], add=True)
                return carry

            lax.fori_loop(0, R2, chunk, 0)
        plsc.subcore_barrier()
        for c in range(2):
            @pl.when(cid == c)
            def _():
                sl = pl.ds(sid * RPT, RPT)
                pltpu.sync_copy(acc.at[sl], outs[c].at[sl])

    return conv2_kernel


def _tc1(flatten, W1):
    def body(f_ref, w_ref, o_ref):
        o_ref[0] = jnp.dot(f_ref[...], w_ref[0], preferred_element_type=jnp.float32)

    return pl.pallas_call(
        body,
        grid=(NET,),
        in_specs=[pl.BlockSpec((N, D_IN), lambda i: (0, 0)),
                  pl.BlockSpec((1, D_IN, H), lambda i: (i, 0, 0))],
        out_specs=pl.BlockSpec((1, N, H), lambda i: (i, 0, 0)),
        out_shape=jax.ShapeDtypeStruct((NET, N, H), jnp.float32),
    )(flatten, W1)


def _tc2(degA, degB, x1):
    def body(dA, dB, x_ref, zA_ref, zB_ref, di_ref):
        deg = dA[0, 0] + dB[0, 0] + 2.0
        di = lax.rsqrt(deg)
        z = x_ref[0] * di[:, None]
        zA_ref[0] = z[:, :HH]
        zB_ref[0] = z[:, HH:]
        di_ref[0, 0] = di

    return pl.pallas_call(
        body,
        grid=(NET,),
        in_specs=[pl.BlockSpec((1, 1, N), lambda i: (i, 0, 0)),
                  pl.BlockSpec((1, 1, N), lambda i: (i, 0, 0)),
                  pl.BlockSpec((1, N, H), lambda i: (i, 0, 0))],
        out_specs=[pl.BlockSpec((1, N, HH), lambda i: (i, 0, 0)),
                   pl.BlockSpec((1, N, HH), lambda i: (i, 0, 0)),
                   pl.BlockSpec((1, 1, N), lambda i: (i, 0, 0))],
        out_shape=[jax.ShapeDtypeStruct((NET, N, HH), jnp.float32),
                   jax.ShapeDtypeStruct((NET, N, HH), jnp.float32),
                   jax.ShapeDtypeStruct((NET, 1, N), jnp.float32)],
    )(degA, degB, x1)


def _tc3(accA, accB, x1, dinv, b1, gamma, beta, W2):
    def body(aA, aB, x1_ref, di_ref, b1_ref, g_ref, be_ref, w2_ref, y2_ref, z2_ref):
        di = di_ref[0, 0]
        agg = jnp.concatenate([aA[0], aB[0]], axis=1)
        x = agg * di[:, None] + x1_ref[0] * (2.0 * di * di)[:, None] + b1_ref[0, 0][None, :]
        x = jnp.maximum(x, 0.0)
        mean = jnp.mean(x, axis=0, keepdims=True)
        var = jnp.mean((x - mean) ** 2, axis=0, keepdims=True)
        x = (x - mean) * lax.rsqrt(var + 1e-5) * g_ref[0, 0][None, :] + be_ref[0, 0][None, :]
        y2 = jnp.dot(x, w2_ref[0], preferred_element_type=jnp.float32)
        y2_ref[0] = y2
        z2_ref[0] = y2 * di[:, None]

    return pl.pallas_call(
        body,
        grid=(NET,),
        in_specs=[pl.BlockSpec((1, N, HH), lambda i: (i, 0, 0)),
                  pl.BlockSpec((1, N, HH), lambda i: (i, 0, 0)),
                  pl.BlockSpec((1, N, H), lambda i: (i, 0, 0)),
                  pl.BlockSpec((1, 1, N), lambda i: (i, 0, 0)),
                  pl.BlockSpec((1, 1, H), lambda i: (i, 0, 0)),
                  pl.BlockSpec((1, 1, H), lambda i: (i, 0, 0)),
                  pl.BlockSpec((1, 1, H), lambda i: (i, 0, 0)),
                  pl.BlockSpec((1, H, 2), lambda i: (i, 0, 0))],
        out_specs=[pl.BlockSpec((1, N, 2), lambda i: (i, 0, 0)),
                   pl.BlockSpec((1, N, 2), lambda i: (i, 0, 0))],
        out_shape=[jax.ShapeDtypeStruct((NET, N, 2), jnp.float32)] * 2,
        compiler_params=pltpu.CompilerParams(
            vmem_limit_bytes=100 * 1024 * 1024),
    )(accA, accB, x1, dinv, b1, gamma, beta, W2)


def _tc4(a2At, a2Bt, y2t, dinv3, b2c, moeT, gWT, gbc):
    def body(aA, aB, y2_ref, di_ref, b2_ref, moeT_ref, gWT_ref, gb_ref,
             outl, outp):
        logitsT = jnp.dot(gWT_ref[...], moeT_ref[...],
                          preferred_element_type=jnp.float32) + gb_ref[...]
        m = jnp.max(logitsT, axis=0, keepdims=True)
        eg = jnp.exp(logitsT - m)
        gT = eg / jnp.sum(eg, axis=0, keepdims=True)
        accl = jnp.zeros((2, N), jnp.float32)
        accp = jnp.zeros((2, N), jnp.float32)
        for i in range(NET):
            di = di_ref[i]
            o = (aA[i] + aB[i]) * di + y2_ref[i] * (2.0 * di * di) + b2_ref[i]
            mm = jnp.max(o, axis=0, keepdims=True)
            lse = mm + jnp.log(jnp.sum(jnp.exp(o - mm), axis=0, keepdims=True))
            lp = o - lse
            pp = jnp.exp(lp)
            w = gT[i:i + 1]
            accl = accl + w * lp
            accp = accp + w * pp
        outl[...] = accl
        outp[...] = accp

    return pl.pallas_call(
        body,
        out_shape=[jax.ShapeDtypeStruct((2, N), jnp.float32)] * 2,
    )(a2At, a2Bt, y2t, dinv3, b2c, moeT, gWT, gbc)


def kernel(features, moe_features, networks, flatten, W1, b1, gamma, beta,
           W2, b2, gW, gb):
    nets = networks.astype(jnp.int32)
    src = jnp.concatenate(
        [nets[:, 0, :], jnp.zeros((NET, EPAD - E), jnp.int32)], axis=1
    ).reshape(NET, EROWS, LANES)
    dst = jnp.concatenate(
        [nets[:, 1, :], jnp.full((NET, EPAD - E), N, jnp.int32)], axis=1
    ).reshape(NET, EROWS, LANES)
    src_l = [src[i] for i in range(NET)]
    dst_l = [dst[i] for i in range(NET)]
    zerosH = jnp.zeros((RPT, HH), jnp.float32)
    onesoh_l = [jnp.zeros((LANES, HH), jnp.float32).at[:, i].set(1.0)
                for i in range(NET)]

    deg_outs = _build_deg_kernel()(*dst_l, *onesoh_l, zerosH)
    degA = jnp.stack([deg_outs[0][:N, i] for i in range(NET)])[:, None, :]
    degB = jnp.stack([deg_outs[1][:N, i] for i in range(NET)])[:, None, :]

    x1 = _tc1(flatten, W1)
    zA, zB, dinv = _tc2(degA, degB, x1)

    z1_l = [zA[i] for i in range(NET)] + [zB[i] for i in range(NET)]
    a1 = _build_conv1_kernel()(*src_l, *dst_l, *z1_l, zerosH)
    accA = jnp.stack([a1[i][:N] for i in range(NET)])
    accB = jnp.stack([a1[4 + i][:N] for i in range(NET)])

    y2, z2 = _tc3(accA, accB, x1, dinv, b1[:, None, :], gamma[:, None, :],
                  beta[:, None, :], W2)
    z2pad = jnp.pad(z2, ((0, 0), (0, 0), (0, HH - 2)))
    z2p_l = [jnp.roll(z2pad[i], 2 * i, axis=1) for i in range(NET)]
    a2 = _build_conv2_kernel()(*src_l, *dst_l, *z2p_l, zerosH)
    a2At = jnp.stack([jnp.transpose(a2[0][:N, 2 * i:2 * i + 2])
                      for i in range(NET)])
    a2Bt = jnp.stack([jnp.transpose(a2[1][:N, 2 * i:2 * i + 2])
                      for i in range(NET)])

    res = _tc4(a2At, a2Bt, jnp.transpose(y2, (0, 2, 1)), dinv,
               b2[:, :, None], jnp.transpose(moe_features),
               jnp.transpose(gW), jnp.reshape(gb, (NET, 1)))
    return (jnp.transpose(res[0]), jnp.transpose(res[1]))


# speedup vs baseline: 18.5193x; 1.2989x over previous
import functools

import jax
import jax.numpy as jnp
from jax import lax
from jax.experimental import pallas as pl
from jax.experimental.pallas import tpu as pltpu
from jax.experimental.pallas import tpu_sc as plsc

N = 10000
E = 320000
NET = 4
D_IN = 128
H = 64
HH = H // 2

LANES = 128
EROWS = 2560
EPAD = EROWS * LANES
R1 = EROWS // 16
R2 = EROWS // 32
NPAD = 10112
RPT = NPAD // 16

def _mesh():
    return plsc.VectorSubcoreMesh(core_axis_name="c", subcore_axis_name="s")


def _agg_edges(ztab, acc, src_v, dst_v, rows0, rows1, gs0, gs1, nrows):
    pltpu.async_copy(ztab.at[src_v.at[0]], rows0, gs0)

    def body(jj, carry):
        j0 = 2 * jj
        pltpu.async_copy(ztab.at[src_v.at[j0 + 1]], rows1, gs1)
        pltpu.make_async_copy(ztab.at[src_v.at[j0]], rows0, gs0).wait()
        pltpu.sync_copy(rows0, acc.at[dst_v.at[j0]], add=True)

        @pl.when(j0 + 2 < nrows)
        def _():
            pltpu.async_copy(ztab.at[src_v.at[j0 + 2]], rows0, gs0)

        pltpu.make_async_copy(ztab.at[src_v.at[j0 + 1]], rows1, gs1).wait()
        pltpu.sync_copy(rows1, acc.at[dst_v.at[j0 + 1]], add=True)
        return carry

    lax.fori_loop(0, nrows // 2, body, 0)


@functools.cache
def _build_deg_kernel():
    @functools.partial(
        pl.kernel,
        out_type=[jax.ShapeDtypeStruct((NPAD, HH), jnp.float32)] * 2,
        mesh=_mesh(),
        compiler_params=pltpu.CompilerParams(use_tc_tiling_on_sc=False),
        scratch_types=[
            pltpu.VMEM((R2, LANES), jnp.int32),
            pltpu.VMEM((LANES, HH), jnp.float32),
            pltpu.VMEM((RPT, HH), jnp.float32),
            pltpu.VMEM_SHARED((NPAD, HH), jnp.float32),
        ],
    )
    def deg_kernel(*refs):
        dsts = refs[0:4]
        onesoh = refs[4:8]
        zeros_h = refs[8]
        outs = refs[9:11]
        idx_v, ones_v, zb, acc = refs[11:15]
        cid = lax.axis_index("c")
        sid = lax.axis_index("s")
        wid = cid * 16 + sid
        pltpu.sync_copy(zeros_h, zb)
        pltpu.sync_copy(zb, acc.at[pl.ds(sid * RPT, RPT)])
        plsc.subcore_barrier()
        for ni in range(NET):
            pltpu.sync_copy(onesoh[ni], ones_v)
            pltpu.sync_copy(dsts[ni].at[pl.ds(wid * R2, R2)], idx_v)

            def chunk(j, carry):
                pltpu.sync_copy(ones_v, acc.at[idx_v.at[j]], add=True)
                return carry

            lax.fori_loop(0, R2, chunk, 0)
        plsc.subcore_barrier()
        for c in range(2):
            @pl.when(cid == c)
            def _():
                sl = pl.ds(sid * RPT, RPT)
                pltpu.sync_copy(acc.at[sl], outs[c].at[sl])

    return deg_kernel


@functools.cache
def _build_conv1_kernel():
    @functools.partial(
        pl.kernel,
        out_type=[jax.ShapeDtypeStruct((NPAD, HH), jnp.float32)] * 8,
        mesh=_mesh(),
        compiler_params=pltpu.CompilerParams(use_tc_tiling_on_sc=False),
        scratch_types=[
            pltpu.VMEM((R1, LANES), jnp.int32),
            pltpu.VMEM((R1, LANES), jnp.int32),
            pltpu.VMEM((LANES, HH), jnp.float32),
            pltpu.VMEM((LANES, HH), jnp.float32),
            pltpu.VMEM((RPT, HH), jnp.float32),
            pltpu.SemaphoreType.DMA,
            pltpu.SemaphoreType.DMA,
        ] + [pltpu.VMEM_SHARED((NPAD, HH), jnp.float32)] * 2,
    )
    def conv1_kernel(*refs):
        srcs = refs[0:4]
        dsts = refs[4:8]
        ztabs = refs[8:16]
        zeros_h = refs[16]
        outs = refs[17:25]
        src_v, dst_v, rows0, rows1, zb, gs0, gs1 = refs[25:32]
        accs = refs[32:34]
        cid = lax.axis_index("c")
        sid = lax.axis_index("s")
        pltpu.sync_copy(zeros_h, zb)
        for g in range(2):
            for k in range(2):
                pltpu.sync_copy(zb, accs[k].at[pl.ds(sid * RPT, RPT)])
            plsc.subcore_barrier()
            for c in range(2):
                @pl.when(cid == c)
                def _():
                    for k in range(2):
                        ni = 2 * g + k
                        pltpu.sync_copy(srcs[ni].at[pl.ds(sid * R1, R1)], src_v)
                        pltpu.sync_copy(dsts[ni].at[pl.ds(sid * R1, R1)], dst_v)
                        _agg_edges(ztabs[c * 4 + ni], accs[k], src_v, dst_v,
                                   rows0, rows1, gs0, gs1, R1)
            plsc.subcore_barrier()
            for c in range(2):
                @pl.when(cid == c)
                def _():
                    for k in range(2):
                        ni = 2 * g + k
                        sl = pl.ds(sid * RPT, RPT)
                        pltpu.sync_copy(accs[k].at[sl], outs[c * 4 + ni].at[sl])

    return conv1_kernel


@functools.cache
def _build_conv2_kernel():
    @functools.partial(
        pl.kernel,
        out_type=[jax.ShapeDtypeStruct((NPAD, HH), jnp.float32)] * 2,
        mesh=_mesh(),
        compiler_params=pltpu.CompilerParams(use_tc_tiling_on_sc=False),
        scratch_types=[
            pltpu.VMEM((R2, LANES), jnp.int32),
            pltpu.VMEM((R2, LANES), jnp.int32),
            pltpu.VMEM((LANES, HH), jnp.float32),
            pltpu.VMEM((LANES, HH), jnp.float32),
            pltpu.VMEM((RPT, HH), jnp.float32),
            pltpu.SemaphoreType.DMA,
            pltpu.SemaphoreType.DMA,
            pltpu.VMEM_SHARED((NPAD, HH), jnp.float32),
        ],
    )
    def conv2_kernel(*refs):
        srcs = refs[0:4]
        dsts = refs[4:8]
        ztabs = refs[8:12]
        zeros_h = refs[12]
        outs = refs[13:15]
        src_v, dst_v, rows0, rows1, zb, gs0, gs1, acc = refs[15:23]
        cid = lax.axis_index("c")
        sid = lax.axis_index("s")
        wid = cid * 16 + sid
        pltpu.sync_copy(zeros_h, zb)
        pltpu.sync_copy(zb, acc.at[pl.ds(sid * RPT, RPT)])
        plsc.subcore_barrier()
        for ni in range(NET):
            pltpu.sync_copy(srcs[ni].at[pl.ds(wid * R2, R2)], src_v)
            pltpu.sync_copy(dsts[ni].at[pl.ds(wid * R2, R2)], dst_v)
            _agg_edges(ztabs[ni], acc, src_v, dst_v,
                       rows0, rows1, gs0, gs1, R2)
        plsc.subcore_barrier()
        for c in range(2):
            @pl.when(cid == c)
            def _():
                sl = pl.ds(sid * RPT, RPT)
                pltpu.sync_copy(acc.at[sl], outs[c].at[sl])

    return conv2_kernel


def _tc1(flatten, W1):
    def body(f_ref, w_ref, o_ref):
        o_ref[0] = jnp.dot(f_ref[...], w_ref[0], preferred_element_type=jnp.float32)

    return pl.pallas_call(
        body,
        grid=(NET,),
        in_specs=[pl.BlockSpec((N, D_IN), lambda i: (0, 0)),
                  pl.BlockSpec((1, D_IN, H), lambda i: (i, 0, 0))],
        out_specs=pl.BlockSpec((1, N, H), lambda i: (i, 0, 0)),
        out_shape=jax.ShapeDtypeStruct((NET, N, H), jnp.float32),
    )(flatten, W1)


def _tc2(degA, degB, x1):
    def body(dA, dB, x_ref, zA_ref, zB_ref, di_ref):
        deg = dA[0, 0] + dB[0, 0] + 2.0
        di = lax.rsqrt(deg)
        z = x_ref[0] * di[:, None]
        zA_ref[0] = z[:, :HH]
        zB_ref[0] = z[:, HH:]
        di_ref[0, 0] = di

    return pl.pallas_call(
        body,
        grid=(NET,),
        in_specs=[pl.BlockSpec((1, 1, N), lambda i: (i, 0, 0)),
                  pl.BlockSpec((1, 1, N), lambda i: (i, 0, 0)),
                  pl.BlockSpec((1, N, H), lambda i: (i, 0, 0))],
        out_specs=[pl.BlockSpec((1, N, HH), lambda i: (i, 0, 0)),
                   pl.BlockSpec((1, N, HH), lambda i: (i, 0, 0)),
                   pl.BlockSpec((1, 1, N), lambda i: (i, 0, 0))],
        out_shape=[jax.ShapeDtypeStruct((NET, N, HH), jnp.float32),
                   jax.ShapeDtypeStruct((NET, N, HH), jnp.float32),
                   jax.ShapeDtypeStruct((NET, 1, N), jnp.float32)],
    )(degA, degB, x1)


def _tc3(accA, accB, x1, dinv, b1, gamma, beta, W2):
    def body(aA, aB, x1_ref, di_ref, b1_ref, g_ref, be_ref, w2_ref, y2_ref, z2_ref):
        di = di_ref[0, 0]
        agg = jnp.concatenate([aA[0], aB[0]], axis=1)
        x = agg * di[:, None] + x1_ref[0] * (2.0 * di * di)[:, None] + b1_ref[0, 0][None, :]
        x = jnp.maximum(x, 0.0)
        mean = jnp.mean(x, axis=0, keepdims=True)
        var = jnp.mean((x - mean) ** 2, axis=0, keepdims=True)
        x = (x - mean) * lax.rsqrt(var + 1e-5) * g_ref[0, 0][None, :] + be_ref[0, 0][None, :]
        y2 = jnp.dot(x, w2_ref[0], preferred_element_type=jnp.float32)
        y2_ref[0] = y2
        z2_ref[0] = y2 * di[:, None]

    return pl.pallas_call(
        body,
        grid=(NET,),
        in_specs=[pl.BlockSpec((1, N, HH), lambda i: (i, 0, 0)),
                  pl.BlockSpec((1, N, HH), lambda i: (i, 0, 0)),
                  pl.BlockSpec((1, N, H), lambda i: (i, 0, 0)),
                  pl.BlockSpec((1, 1, N), lambda i: (i, 0, 0)),
                  pl.BlockSpec((1, 1, H), lambda i: (i, 0, 0)),
                  pl.BlockSpec((1, 1, H), lambda i: (i, 0, 0)),
                  pl.BlockSpec((1, 1, H), lambda i: (i, 0, 0)),
                  pl.BlockSpec((1, H, 2), lambda i: (i, 0, 0))],
        out_specs=[pl.BlockSpec((1, N, 2), lambda i: (i, 0, 0)),
                   pl.BlockSpec((1, N, 2), lambda i: (i, 0, 0))],
        out_shape=[jax.ShapeDtypeStruct((NET, N, 2), jnp.float32)] * 2,
        compiler_params=pltpu.CompilerParams(
            vmem_limit_bytes=100 * 1024 * 1024),
    )(accA, accB, x1, dinv, b1, gamma, beta, W2)


def _tc4(a2At, a2Bt, y2t, dinv3, b2c, moeT, gWT, gbc):
    def body(aA, aB, y2_ref, di_ref, b2_ref, moeT_ref, gWT_ref, gb_ref,
             outl, outp):
        logitsT = jnp.dot(gWT_ref[...], moeT_ref[...],
                          preferred_element_type=jnp.float32) + gb_ref[...]
        m = jnp.max(logitsT, axis=0, keepdims=True)
        eg = jnp.exp(logitsT - m)
        gT = eg / jnp.sum(eg, axis=0, keepdims=True)
        accl = jnp.zeros((2, N), jnp.float32)
        accp = jnp.zeros((2, N), jnp.float32)
        for i in range(NET):
            di = di_ref[i]
            o = (aA[i] + aB[i]) * di + y2_ref[i] * (2.0 * di * di) + b2_ref[i]
            mm = jnp.max(o, axis=0, keepdims=True)
            lse = mm + jnp.log(jnp.sum(jnp.exp(o - mm), axis=0, keepdims=True))
            lp = o - lse
            pp = jnp.exp(lp)
            w = gT[i:i + 1]
            accl = accl + w * lp
            accp = accp + w * pp
        outl[...] = accl
        outp[...] = accp

    return pl.pallas_call(
        body,
        out_shape=[jax.ShapeDtypeStruct((2, N), jnp.float32)] * 2,
    )(a2At, a2Bt, y2t, dinv3, b2c, moeT, gWT, gbc)


def kernel(features, moe_features, networks, flatten, W1, b1, gamma, beta,
           W2, b2, gW, gb):
    nets = networks.astype(jnp.int32)
    src = jnp.concatenate(
        [nets[:, 0, :], jnp.zeros((NET, EPAD - E), jnp.int32)], axis=1
    ).reshape(NET, EROWS, LANES)
    dst = jnp.concatenate(
        [nets[:, 1, :], jnp.full((NET, EPAD - E), N, jnp.int32)], axis=1
    ).reshape(NET, EROWS, LANES)
    src_l = [src[i] for i in range(NET)]
    dst_l = [dst[i] for i in range(NET)]
    zerosH = jnp.zeros((RPT, HH), jnp.float32)
    onesoh_l = [jnp.zeros((LANES, HH), jnp.float32).at[:, i].set(1.0)
                for i in range(NET)]

    deg_outs = _build_deg_kernel()(*dst_l, *onesoh_l, zerosH)
    degA = jnp.stack([deg_outs[0][:N, i] for i in range(NET)])[:, None, :]
    degB = jnp.stack([deg_outs[1][:N, i] for i in range(NET)])[:, None, :]

    x1 = _tc1(flatten, W1)
    zA, zB, dinv = _tc2(degA, degB, x1)

    z1_l = [zA[i] for i in range(NET)] + [zB[i] for i in range(NET)]
    a1 = _build_conv1_kernel()(*src_l, *dst_l, *z1_l, zerosH)
    accA = jnp.stack([a1[i][:N] for i in range(NET)])
    accB = jnp.stack([a1[4 + i][:N] for i in range(NET)])

    y2, z2 = _tc3(accA, accB, x1, dinv, b1[:, None, :], gamma[:, None, :],
                  beta[:, None, :], W2)
    z2pad = jnp.pad(z2, ((0, 0), (0, 0), (0, HH - 2)))
    z2p_l = [jnp.roll(z2pad[i], 2 * i, axis=1) for i in range(NET)]
    a2 = _build_conv2_kernel()(*src_l, *dst_l, *z2p_l, zerosH)
    a2At = jnp.stack([jnp.transpose(a2[0][:N, 2 * i:2 * i + 2])
                      for i in range(NET)])
    a2Bt = jnp.stack([jnp.transpose(a2[1][:N, 2 * i:2 * i + 2])
                      for i in range(NET)])

    res = _tc4(a2At, a2Bt, jnp.transpose(y2, (0, 2, 1)), dinv,
               b2[:, :, None], jnp.transpose(moe_features),
               jnp.transpose(gW), jnp.reshape(gb, (NET, 1)))
    return (jnp.transpose(res[0]), jnp.transpose(res[1]))
